# Initial kernel scaffold; baseline (speedup 1.0000x reference)
import jax
import jax.numpy as jnp
from jax.experimental import pallas as pl


def kernel(h, edge_index, W1, W2, mlp_bn_gamma, mlp_bn_beta, apply_bn_gamma, apply_bn_beta, out_bn_gamma, out_bn_beta):
    raise NotImplementedError("write your pallas kernel here")


# speedup vs baseline: 8.4969x; 8.4969x over previous
import functools

import jax
import jax.numpy as jnp
from jax import lax
from jax.experimental import pallas as pl
from jax.experimental.pallas import tpu as pltpu
from jax.experimental.pallas import tpu_sc as plsc

_N = 10000
_D = 128
_E = 320000
_L = 3

_NC = 2
_NS = 16
_NW = _NC * _NS
_EPW = _E // _NW
_CH = 80
_NCH = _EPW // _CH
_NPAD = _NS * 640
_ROWS_PT = _NPAD // _NS
_ZR = 40

_mesh = plsc.VectorSubcoreMesh(core_axis_name="c", subcore_axis_name="s")


@functools.partial(
    pl.kernel,
    out_type=jax.ShapeDtypeStruct((_NC, _NPAD, _D), jnp.float32),
    mesh=_mesh,
    scratch_types=[
        pltpu.VMEM((2, _CH), jnp.int32),
        pltpu.VMEM((2, _CH), jnp.int32),
        pltpu.VMEM((_CH, _D), jnp.float32),
        pltpu.VMEM((_CH, _D), jnp.float32),
        pltpu.VMEM((_ZR, _D), jnp.float32),
        pltpu.VMEM_SHARED((_NPAD, _D), jnp.float32),
        pltpu.SemaphoreType.DMA,
        pltpu.SemaphoreType.DMA,
        pltpu.SemaphoreType.DMA,
        pltpu.SemaphoreType.DMA,
    ],
)
def _segsum_sc(h_hbm, idx_hbm, out_hbm, ib0, ib1, rb0, rb1, zb,
               acc, s0, s1, si0, si1):
    c = lax.axis_index("c")
    s = lax.axis_index("s")
    w = c * _NS + s

    zvec = jnp.zeros((16,), jnp.float32)

    def _zstore(i, carry):
        zb[i // (_D // 16), pl.ds((i % (_D // 16)) * 16, 16)] = zvec
        return carry

    lax.fori_loop(0, _ZR * (_D // 16), _zstore, 0)

    def _zcopy(i, carry):
        pltpu.sync_copy(zb, acc.at[pl.ds(s * 640 + i * _ZR, _ZR)])
        return carry

    lax.fori_loop(0, 640 // _ZR, _zcopy, 0)
    plsc.subcore_barrier()

    pltpu.sync_copy(idx_hbm.at[w, 0], ib0)
    pltpu.async_copy(h_hbm.at[ib0.at[0]], rb0, s0)
    pltpu.async_copy(idx_hbm.at[w, 1], ib1, si1)

    def _pair(i, carry):
        j = 2 * i
        pltpu.make_async_copy(idx_hbm.at[w, j + 1], ib1, si1).wait()
        pltpu.async_copy(h_hbm.at[ib1.at[0]], rb1, s1)
        pltpu.make_async_copy(h_hbm.at[ib0.at[0]], rb0, s0).wait()
        pltpu.sync_copy(rb0, acc.at[ib0.at[1]], add=True)
        pltpu.async_copy(idx_hbm.at[w, j + 2], ib0, si0)
        pltpu.make_async_copy(idx_hbm.at[w, j + 2], ib0, si0).wait()
        pltpu.async_copy(h_hbm.at[ib0.at[0]], rb0, s0)
        pltpu.make_async_copy(h_hbm.at[ib1.at[0]], rb1, s1).wait()
        pltpu.sync_copy(rb1, acc.at[ib1.at[1]], add=True)

        @pl.when(j + 3 < _NCH)
        def _():
            pltpu.async_copy(idx_hbm.at[w, j + 3], ib1, si1)

        return carry

    lax.fori_loop(0, (_NCH - 1) // 2, _pair, 0)
    pltpu.make_async_copy(h_hbm.at[ib0.at[0]], rb0, s0).wait()
    pltpu.sync_copy(rb0, acc.at[ib0.at[1]], add=True)

    plsc.subcore_barrier()
    pltpu.sync_copy(acc.at[pl.ds(s * _ROWS_PT, _ROWS_PT)],
                    out_hbm.at[c, pl.ds(s * _ROWS_PT, _ROWS_PT)])


def _bn(z, gamma, beta, relu):
    mean = jnp.mean(z, axis=0, keepdims=True)
    zc = z - mean
    var = jnp.mean(zc * zc, axis=0, keepdims=True)
    out = gamma * zc * lax.rsqrt(var + 1e-5) + beta
    return jnp.maximum(out, 0.0) if relu else out


def _tc_layer_body(h_ref, p_ref, w1t_ref, w2t_ref, g1_ref, b1_ref, ga_ref,
                   ba_ref, go_ref, bo_ref, out_ref, *, relu_out):
    x = h_ref[...] + p_ref[0, :_N] + p_ref[1, :_N]
    z = jnp.dot(x, w1t_ref[...], preferred_element_type=jnp.float32)
    z = _bn(z, g1_ref[...], b1_ref[...], relu=True)
    z = jnp.dot(z, w2t_ref[...], preferred_element_type=jnp.float32)
    z = _bn(z, ga_ref[...], ba_ref[...], relu=True)
    out_ref[...] = _bn(z, go_ref[...], bo_ref[...], relu=relu_out)


def _tc_layer(relu_out):
    return pl.pallas_call(
        functools.partial(_tc_layer_body, relu_out=relu_out),
        out_shape=jax.ShapeDtypeStruct((_N, _D), jnp.float32),
    )


def kernel(h, edge_index, W1, W2, mlp_bn_gamma, mlp_bn_beta, apply_bn_gamma,
           apply_bn_beta, out_bn_gamma, out_bn_beta):
    idx = jnp.stack([edge_index[0].reshape(_NW, _NCH, _CH),
                     edge_index[1].reshape(_NW, _NCH, _CH)], axis=2)
    for i in range(_L):
        parts = _segsum_sc(h, idx)
        h = _tc_layer(i != _L - 1)(
            h, parts,
            W1[i].T, W2[i].T,
            mlp_bn_gamma[i].reshape(1, _D), mlp_bn_beta[i].reshape(1, _D),
            apply_bn_gamma[i].reshape(1, _D), apply_bn_beta[i].reshape(1, _D),
            out_bn_gamma[i].reshape(1, _D), out_bn_beta[i].reshape(1, _D),
        )
    return h
